# Initial kernel scaffold; baseline (speedup 1.0000x reference)
import jax
import jax.numpy as jnp
from jax.experimental import pallas as pl


def kernel(y, hidden, centers):
    raise NotImplementedError("write your pallas kernel here")


# speedup vs baseline: 2.0279x; 2.0279x over previous
import functools

import jax
import jax.numpy as jnp
from jax import lax
from jax.experimental import pallas as pl
from jax.experimental.pallas import tpu as pltpu


def _round_up(x, m):
    return (x + m - 1) // m * m


def _issue_gather(y_sref, centers_hbm, gbuf, sems, base, slot, tile_b):
    for r in range(tile_b):
        row = y_sref[base + r]
        pltpu.make_async_copy(centers_hbm.at[pl.ds(row, 1)],
                              gbuf.at[slot, pl.ds(r, 1)],
                              sems.at[slot]).start()


def _wait_gather(centers_hbm, gbuf, sems, slot, tile_b):
    pltpu.make_async_copy(centers_hbm.at[pl.ds(0, tile_b)],
                          gbuf.at[slot],
                          sems.at[slot]).wait()


def _loss_kernel(y_sref, hidden_ref, centers_hbm, out_ref, gbuf, sems, *,
                 lambda_c, tile_b, batch, nbj):
    c = pl.program_id(0)
    j = pl.program_id(1)
    slot = lax.rem(j, 2)
    base = (c * nbj + j) * tile_b

    @pl.when(j == 0)
    def _():
        out_ref[...] = jnp.zeros_like(out_ref)
        _issue_gather(y_sref, centers_hbm, gbuf, sems, base, 0, tile_b)

    @pl.when(j + 1 < nbj)
    def _():
        _issue_gather(y_sref, centers_hbm, gbuf, sems, base + tile_b,
                      lax.rem(j + 1, 2), tile_b)

    _wait_gather(centers_hbm, gbuf, sems, slot, tile_b)

    e = gbuf[slot].astype(jnp.float32)
    h = hidden_ref[...].astype(jnp.float32)

    c_sq = jnp.sum(e * e, axis=1, keepdims=True)
    f_sq = jnp.sum(h * h, axis=1, keepdims=True)
    inv_c = jnp.minimum(lax.rsqrt(c_sq), 1.0)
    inv_f = jnp.minimum(lax.rsqrt(f_sq), 1.0)

    d = h - e
    row_sq = jnp.sum(d * d, axis=1, keepdims=True)

    rows = base + lax.broadcasted_iota(jnp.int32, (tile_b, 1), 0)
    valid = (rows < batch).astype(jnp.float32)

    contrib = row_sq * inv_f * inv_c * valid
    tile_sum = jnp.sum(contrib, axis=0, keepdims=True)

    out_ref[...] += (lambda_c / 2.0 / batch) * tile_sum.reshape(1, 1, 1)


def _center_loss(y, hidden, centers, lambda_c=1.0):
    B, D = hidden.shape
    C = centers.shape[0]

    n_cores = 2
    tb = min(256, _round_up(max(B, 8), 8))
    b_pad = _round_up(B, n_cores * tb)
    nbj = b_pad // (n_cores * tb)

    y_c = jnp.clip(y.astype(jnp.int32).reshape(-1), 0, C - 1)
    if b_pad != B:
        y_c = jnp.pad(y_c, (0, b_pad - B))
        hidden = jnp.pad(hidden, ((0, b_pad - B), (0, 0)))

    _kbody = functools.partial(_loss_kernel, lambda_c=float(lambda_c),
                               tile_b=tb, batch=B, nbj=nbj)
    out = pl.pallas_call(
        _kbody,
        out_shape=jax.ShapeDtypeStruct((n_cores, 1, 1), jnp.float32),
        grid_spec=pltpu.PrefetchScalarGridSpec(
            num_scalar_prefetch=1,
            grid=(n_cores, nbj),
            in_specs=[
                pl.BlockSpec((tb, D), lambda c, j, y_s: (c * nbj + j, 0)),
                pl.BlockSpec(memory_space=pl.ANY),
            ],
            out_specs=pl.BlockSpec((1, 1, 1), lambda c, j, y_s: (c, 0, 0)),
            scratch_shapes=[
                pltpu.VMEM((2, tb, D), centers.dtype),
                pltpu.SemaphoreType.DMA((2,)),
            ],
        ),
        compiler_params=pltpu.CompilerParams(
            dimension_semantics=("parallel", "arbitrary"),
            vmem_limit_bytes=64 << 20,
        ),
    )(y_c, hidden, centers)
    return out[0, 0, 0] + out[1, 0, 0]


def kernel(y, hidden, centers):
    return _center_loss(y, hidden, centers, lambda_c=1.0)
